# Initial kernel scaffold; baseline (speedup 1.0000x reference)
import jax
import jax.numpy as jnp
from jax.experimental import pallas as pl


def kernel(x, edge_index, W1l, b1, W1r, W2l, b2, W2r):
    raise NotImplementedError("write your pallas kernel here")


# speedup vs baseline: 7.0316x; 7.0316x over previous
import functools

import jax
import jax.numpy as jnp
from jax import lax
from jax.experimental import pallas as pl
from jax.experimental.pallas import tpu as pltpu
from jax.experimental.pallas import tpu_sc as plsc

N = 10000
E = 320000
F = 128
H = 64
O = 32

NC, NS = 2, 16
NW = NC * NS
NP = 10240
EP = 327680
PT = EP // NW
CH = 1024
SUB = CH // 128
NCHUNK = PT // CH
ZR = NP // NS
RB = 1024


def _mm2_body(x_ref, wl_ref, wr_ref, b_ref, p_ref, r_ref):
    xb = x_ref[...]
    p_ref[...] = jnp.dot(xb, wl_ref[...], preferred_element_type=jnp.float32)
    r_ref[...] = (jnp.dot(xb, wr_ref[...], preferred_element_type=jnp.float32)
                  + b_ref[...])


def _project2(xp, wlT, wrT, b, d_in, d_out):
    return pl.pallas_call(
        _mm2_body,
        grid=(NP // RB,),
        in_specs=[
            pl.BlockSpec((RB, d_in), lambda i: (i, 0)),
            pl.BlockSpec((d_in, d_out), lambda i: (0, 0)),
            pl.BlockSpec((d_in, d_out), lambda i: (0, 0)),
            pl.BlockSpec((1, d_out), lambda i: (0, 0)),
        ],
        out_specs=[
            pl.BlockSpec((RB, d_out), lambda i: (i, 0)),
            pl.BlockSpec((RB, d_out), lambda i: (i, 0)),
        ],
        out_shape=[
            jax.ShapeDtypeStruct((NP, d_out), jnp.float32),
            jax.ShapeDtypeStruct((NP, d_out), jnp.float32),
        ],
    )(xp, wlT, wrT, b.reshape(1, d_out))


def _combine_mm2_body(parts_ref, cnts_ref, r_ref, wl_ref, wr_ref, b_ref,
                      p_ref, r2_ref):
    agg = parts_ref[0] + parts_ref[1]
    cnt = cnts_ref[0, :, 0:1] + cnts_ref[1, :, 0:1]
    mean = agg / jnp.maximum(cnt, 1.0)
    h = jnp.maximum(mean + r_ref[...], 0.0)
    p_ref[...] = jnp.dot(h, wl_ref[...], preferred_element_type=jnp.float32)
    r2_ref[...] = (jnp.dot(h, wr_ref[...], preferred_element_type=jnp.float32)
                   + b_ref[...])


def _combine_project2(parts, cnts, r, wlT, wrT, b, d_in, d_out):
    return pl.pallas_call(
        _combine_mm2_body,
        grid=(NP // RB,),
        in_specs=[
            pl.BlockSpec((NC, RB, d_in), lambda i: (0, i, 0)),
            pl.BlockSpec((NC, RB, 8), lambda i: (0, i, 0)),
            pl.BlockSpec((RB, d_in), lambda i: (i, 0)),
            pl.BlockSpec((d_in, d_out), lambda i: (0, 0)),
            pl.BlockSpec((d_in, d_out), lambda i: (0, 0)),
            pl.BlockSpec((1, d_out), lambda i: (0, 0)),
        ],
        out_specs=[
            pl.BlockSpec((RB, d_out), lambda i: (i, 0)),
            pl.BlockSpec((RB, d_out), lambda i: (i, 0)),
        ],
        out_shape=[
            jax.ShapeDtypeStruct((NP, d_out), jnp.float32),
            jax.ShapeDtypeStruct((NP, d_out), jnp.float32),
        ],
    )(parts, cnts, r, wlT, wrT, b.reshape(1, d_out))


def _final_body(parts_ref, cnts_ref, r_ref, out_ref):
    agg = parts_ref[0] + parts_ref[1]
    cnt = cnts_ref[0, :, 0:1] + cnts_ref[1, :, 0:1]
    out_ref[...] = agg / jnp.maximum(cnt, 1.0) + r_ref[...]


def _final_combine(parts, cnts, r, d):
    return pl.pallas_call(
        _final_body,
        grid=(NP // RB,),
        in_specs=[
            pl.BlockSpec((NC, RB, d), lambda i: (0, i, 0)),
            pl.BlockSpec((NC, RB, 8), lambda i: (0, i, 0)),
            pl.BlockSpec((RB, d), lambda i: (i, 0)),
        ],
        out_specs=pl.BlockSpec((RB, d), lambda i: (i, 0)),
        out_shape=jax.ShapeDtypeStruct((NP, d), jnp.float32),
    )(parts, cnts, r)


def _make_edge_scatter(d, with_counts):
    mesh = plsc.VectorSubcoreMesh(core_axis_name="c", subcore_axis_name="s",
                                  num_cores=NC, num_subcores=NS)

    out_type = [jax.ShapeDtypeStruct((NC, NP, d), jnp.float32)]
    scratch = [
        pltpu.VMEM((SUB, 128), jnp.int32),
        pltpu.VMEM((SUB, 128), jnp.int32),
        pltpu.VMEM((CH, d), jnp.float32),
        pltpu.VMEM_SHARED((NP, d), jnp.float32),
        pltpu.SemaphoreType.DMA,
    ]
    if with_counts:
        out_type.append(jax.ShapeDtypeStruct((NC, NP, 8), jnp.float32))
        scratch += [
            pltpu.VMEM((128, 8), jnp.float32),
            pltpu.VMEM_SHARED((NP, 8), jnp.float32),
        ]

    def body(table, src2d, dst2d, zeros_d, zeros_c, ones, *rest):
        if with_counts:
            (out, cnt_out, src_v, dst_v, rows_v, acc, sem,
             ones_v, cnt_acc) = rest
        else:
            out, src_v, dst_v, rows_v, acc, sem = rest
        cid = lax.axis_index("c")
        sid = lax.axis_index("s")
        tile = cid * NS + sid

        zlo = sid * ZR
        pltpu.sync_copy(zeros_d.at[pl.ds(zlo, ZR)], acc.at[pl.ds(zlo, ZR)])
        if with_counts:
            pltpu.sync_copy(zeros_c.at[pl.ds(zlo, ZR)],
                            cnt_acc.at[pl.ds(zlo, ZR)])
            pltpu.sync_copy(ones, ones_v)
        plsc.subcore_barrier()

        def chunk(i, carry):
            row0 = tile * (PT // 128) + i * SUB
            pltpu.sync_copy(src2d.at[pl.ds(row0, SUB)], src_v)
            pltpu.sync_copy(dst2d.at[pl.ds(row0, SUB)], dst_v)
            copies = [
                pltpu.async_copy(table.at[src_v.at[j]],
                                 rows_v.at[pl.ds(j * 128, 128)], sem)
                for j in range(SUB)
            ]
            for c in copies:
                c.wait()
            for j in range(SUB):
                pltpu.sync_copy(rows_v.at[pl.ds(j * 128, 128)],
                                acc.at[dst_v.at[j]], add=True)
            if with_counts:
                for j in range(SUB):
                    pltpu.sync_copy(ones_v, cnt_acc.at[dst_v.at[j]], add=True)
            return carry

        lax.fori_loop(0, NCHUNK, chunk, None)
        plsc.subcore_barrier()

        pltpu.sync_copy(acc.at[pl.ds(zlo, ZR)],
                        out.at[cid, pl.ds(zlo, ZR)])
        if with_counts:
            pltpu.sync_copy(cnt_acc.at[pl.ds(zlo, ZR)],
                            cnt_out.at[cid, pl.ds(zlo, ZR)])

    return pl.kernel(body, out_type=out_type, mesh=mesh,
                     scratch_types=scratch,
                     compiler_params=pltpu.CompilerParams(
                         use_tc_tiling_on_sc=False))


_edge_scatter_cnt = _make_edge_scatter(H, True)
_edge_scatter = _make_edge_scatter(O, False)


def kernel(x, edge_index, W1l, b1, W1r, W2l, b2, W2r):
    src = edge_index[0].astype(jnp.int32)
    dst = edge_index[1].astype(jnp.int32)
    pad = EP - E
    src2d = jnp.concatenate(
        [src, jnp.full((pad,), N, jnp.int32)]).reshape(EP // 128, 128)
    dst2d = jnp.concatenate(
        [dst, jnp.full((pad,), N, jnp.int32)]).reshape(EP // 128, 128)
    xp = jnp.pad(x, ((0, NP - N), (0, 0)))

    zeros_h = jnp.zeros((NP, H), jnp.float32)
    zeros_o = jnp.zeros((NP, O), jnp.float32)
    zeros_c = jnp.zeros((NP, 8), jnp.float32)
    ones = jnp.ones((128, 8), jnp.float32)

    p1, r1 = _project2(xp, W1l.T, W1r.T, b1, F, H)
    part1, cnts = _edge_scatter_cnt(p1, src2d, dst2d, zeros_h, zeros_c, ones)
    p2, r2 = _combine_project2(part1, cnts, r1, W2l.T, W2r.T, b2, H, O)
    part2, = _edge_scatter(p2, src2d, dst2d, zeros_o, zeros_c, ones)
    out = _final_combine(part2, cnts, r2, O)
    return out[:N]


# speedup vs baseline: 13.7703x; 1.9583x over previous
import jax
import jax.numpy as jnp
from jax import lax
from jax.experimental import pallas as pl
from jax.experimental.pallas import tpu as pltpu
from jax.experimental.pallas import tpu_sc as plsc

N = 10000
E = 320000
F = 128
H = 64
O = 32

NC, NS = 2, 16
NW = NC * NS
NP = 10240
EP = 327680
PT = EP // NW
CH = 1024
SUB = CH // 128
NCHUNK = PT // CH
ZR = NP // NS
RB = 1024
W1 = H + 8


def _mm2_body(x_ref, wl_ref, wr_ref, b_ref, p_ref, r_ref):
    xb = x_ref[...]
    mm = jnp.dot(xb, wl_ref[...], preferred_element_type=jnp.float32)
    p_ref[...] = jnp.concatenate(
        [mm, jnp.ones((RB, 1), jnp.float32), jnp.zeros((RB, 7), jnp.float32)],
        axis=1)
    r_ref[...] = (jnp.dot(xb, wr_ref[...], preferred_element_type=jnp.float32)
                  + b_ref[...])


def _project2(xp, wlT, wrT, b):
    return pl.pallas_call(
        _mm2_body,
        grid=(NP // RB,),
        in_specs=[
            pl.BlockSpec((RB, F), lambda i: (i, 0)),
            pl.BlockSpec((F, H), lambda i: (0, 0)),
            pl.BlockSpec((F, H), lambda i: (0, 0)),
            pl.BlockSpec((1, H), lambda i: (0, 0)),
        ],
        out_specs=[
            pl.BlockSpec((RB, W1), lambda i: (i, 0)),
            pl.BlockSpec((RB, H), lambda i: (i, 0)),
        ],
        out_shape=[
            jax.ShapeDtypeStruct((NP, W1), jnp.float32),
            jax.ShapeDtypeStruct((NP, H), jnp.float32),
        ],
    )(xp, wlT, wrT, b.reshape(1, H))


def _combine_mm2_body(parts_ref, r_ref, wl_ref, wr_ref, b_ref,
                      p_ref, r2_ref):
    agg = parts_ref[0, :, :H] + parts_ref[1, :, :H]
    cnt = parts_ref[0, :, H:H + 1] + parts_ref[1, :, H:H + 1]
    mean = agg / jnp.maximum(cnt, 1.0)
    h = jnp.maximum(mean + r_ref[...], 0.0)
    p_ref[...] = jnp.dot(h, wl_ref[...], preferred_element_type=jnp.float32)
    r2_ref[...] = (jnp.dot(h, wr_ref[...], preferred_element_type=jnp.float32)
                   + b_ref[...])


def _combine_project2(parts, r, wlT, wrT, b):
    return pl.pallas_call(
        _combine_mm2_body,
        grid=(NP // RB,),
        in_specs=[
            pl.BlockSpec((NC, RB, W1), lambda i: (0, i, 0)),
            pl.BlockSpec((RB, H), lambda i: (i, 0)),
            pl.BlockSpec((H, O), lambda i: (0, 0)),
            pl.BlockSpec((H, O), lambda i: (0, 0)),
            pl.BlockSpec((1, O), lambda i: (0, 0)),
        ],
        out_specs=[
            pl.BlockSpec((RB, O), lambda i: (i, 0)),
            pl.BlockSpec((RB, O), lambda i: (i, 0)),
        ],
        out_shape=[
            jax.ShapeDtypeStruct((NP, O), jnp.float32),
            jax.ShapeDtypeStruct((NP, O), jnp.float32),
        ],
    )(parts, r, wlT, wrT, b.reshape(1, O))


def _final_body(parts_ref, cnts_ref, r_ref, out_ref):
    agg = parts_ref[0] + parts_ref[1]
    cnt = cnts_ref[0, :, H:H + 1] + cnts_ref[1, :, H:H + 1]
    out_ref[...] = agg / jnp.maximum(cnt, 1.0) + r_ref[...]


def _final_combine(parts, cnts, r):
    fb = 1000
    return pl.pallas_call(
        _final_body,
        grid=(N // fb,),
        in_specs=[
            pl.BlockSpec((NC, fb, O), lambda i: (0, i, 0)),
            pl.BlockSpec((NC, fb, W1), lambda i: (0, i, 0)),
            pl.BlockSpec((fb, O), lambda i: (i, 0)),
        ],
        out_specs=pl.BlockSpec((fb, O), lambda i: (i, 0)),
        out_shape=jax.ShapeDtypeStruct((N, O), jnp.float32),
    )(parts, cnts, r)


def _make_edge_scatter(d):
    mesh = plsc.VectorSubcoreMesh(core_axis_name="c", subcore_axis_name="s",
                                  num_cores=NC, num_subcores=NS)

    def body(table, src2d, dst2d, zeros_d, out, src_v, dst_v, rows_v, acc,
             sem):
        cid = lax.axis_index("c")
        sid = lax.axis_index("s")
        tile = cid * NS + sid

        zlo = sid * ZR
        pltpu.sync_copy(zeros_d.at[pl.ds(zlo, ZR)], acc.at[pl.ds(zlo, ZR)])
        plsc.subcore_barrier()

        def chunk(i, carry):
            row0 = tile * (PT // 128) + i * SUB
            pltpu.sync_copy(src2d.at[pl.ds(row0, SUB)], src_v)
            pltpu.sync_copy(dst2d.at[pl.ds(row0, SUB)], dst_v)
            copies = [
                pltpu.async_copy(table.at[src_v.at[j]],
                                 rows_v.at[pl.ds(j * 128, 128)], sem)
                for j in range(SUB)
            ]
            for c in copies:
                c.wait()
            for j in range(SUB):
                pltpu.sync_copy(rows_v.at[pl.ds(j * 128, 128)],
                                acc.at[dst_v.at[j]], add=True)
            return carry

        lax.fori_loop(0, NCHUNK, chunk, None)
        plsc.subcore_barrier()

        pltpu.sync_copy(acc.at[pl.ds(zlo, ZR)],
                        out.at[cid, pl.ds(zlo, ZR)])

    return pl.kernel(
        body,
        out_type=[jax.ShapeDtypeStruct((NC, NP, d), jnp.float32)],
        mesh=mesh,
        scratch_types=[
            pltpu.VMEM((SUB, 128), jnp.int32),
            pltpu.VMEM((SUB, 128), jnp.int32),
            pltpu.VMEM((CH, d), jnp.float32),
            pltpu.VMEM_SHARED((NP, d), jnp.float32),
            pltpu.SemaphoreType.DMA,
        ],
        compiler_params=pltpu.CompilerParams(use_tc_tiling_on_sc=False))


_edge_scatter_l1 = _make_edge_scatter(W1)
_edge_scatter_l2 = _make_edge_scatter(O)


def kernel(x, edge_index, W1l, b1, W1r, W2l, b2, W2r):
    src = edge_index[0].astype(jnp.int32)
    dst = edge_index[1].astype(jnp.int32)
    pad = EP - E
    pad_idx = N + jnp.arange(pad, dtype=jnp.int32) % (NP - N)
    src2d = jnp.concatenate([src, pad_idx]).reshape(EP // 128, 128)
    dst2d = jnp.concatenate([dst, pad_idx]).reshape(EP // 128, 128)
    xp = jnp.pad(x, ((0, NP - N), (0, 0)))

    zeros_1 = jnp.zeros((NP, W1), jnp.float32)
    zeros_2 = jnp.zeros((NP, O), jnp.float32)

    p1, r1 = _project2(xp, W1l.T, W1r.T, b1)
    part1, = _edge_scatter_l1(p1, src2d, dst2d, zeros_1)
    p2, r2 = _combine_project2(part1, r1, W2l.T, W2r.T, b2)
    part2, = _edge_scatter_l2(p2, src2d, dst2d, zeros_2)
    return _final_combine(part2, part1, r2)


# speedup vs baseline: 16.5715x; 1.2034x over previous
import jax
import jax.numpy as jnp
from jax import lax
from jax.experimental import pallas as pl
from jax.experimental.pallas import tpu as pltpu
from jax.experimental.pallas import tpu_sc as plsc

N = 10000
E = 320000
F = 128
H = 64
O = 32

NC, NS = 2, 16
NW = NC * NS
NR = E // 128
RT = NR // NW
NX = NR - RT * NW
ZR = N // NS
RB = 1000
W1 = H + 8


def _mm2_body(x_ref, wl_ref, wr_ref, b_ref, p_ref, r_ref):
    xb = x_ref[...]
    mm = jnp.dot(xb, wl_ref[...], preferred_element_type=jnp.float32)
    p_ref[...] = jnp.concatenate(
        [mm, jnp.ones((RB, 1), jnp.float32), jnp.zeros((RB, 7), jnp.float32)],
        axis=1)
    r_ref[...] = (jnp.dot(xb, wr_ref[...], preferred_element_type=jnp.float32)
                  + b_ref[...])


def _project2(x, wlT, wrT, b):
    return pl.pallas_call(
        _mm2_body,
        grid=(N // RB,),
        in_specs=[
            pl.BlockSpec((RB, F), lambda i: (i, 0)),
            pl.BlockSpec((F, H), lambda i: (0, 0)),
            pl.BlockSpec((F, H), lambda i: (0, 0)),
            pl.BlockSpec((1, H), lambda i: (0, 0)),
        ],
        out_specs=[
            pl.BlockSpec((RB, W1), lambda i: (i, 0)),
            pl.BlockSpec((RB, H), lambda i: (i, 0)),
        ],
        out_shape=[
            jax.ShapeDtypeStruct((N, W1), jnp.float32),
            jax.ShapeDtypeStruct((N, H), jnp.float32),
        ],
    )(x, wlT, wrT, b.reshape(1, H))


def _combine_mm2_body(parts_ref, r_ref, wl_ref, wr_ref, b_ref,
                      p_ref, r2_ref):
    agg = parts_ref[0, :, :H] + parts_ref[1, :, :H]
    cnt = parts_ref[0, :, H:H + 1] + parts_ref[1, :, H:H + 1]
    mean = agg / jnp.maximum(cnt, 1.0)
    h = jnp.maximum(mean + r_ref[...], 0.0)
    p_ref[...] = jnp.dot(h, wl_ref[...], preferred_element_type=jnp.float32)
    r2_ref[...] = (jnp.dot(h, wr_ref[...], preferred_element_type=jnp.float32)
                   + b_ref[...])


def _combine_project2(parts, r, wlT, wrT, b):
    return pl.pallas_call(
        _combine_mm2_body,
        grid=(N // RB,),
        in_specs=[
            pl.BlockSpec((NC, RB, W1), lambda i: (0, i, 0)),
            pl.BlockSpec((RB, H), lambda i: (i, 0)),
            pl.BlockSpec((H, O), lambda i: (0, 0)),
            pl.BlockSpec((H, O), lambda i: (0, 0)),
            pl.BlockSpec((1, O), lambda i: (0, 0)),
        ],
        out_specs=[
            pl.BlockSpec((RB, O), lambda i: (i, 0)),
            pl.BlockSpec((RB, O), lambda i: (i, 0)),
        ],
        out_shape=[
            jax.ShapeDtypeStruct((N, O), jnp.float32),
            jax.ShapeDtypeStruct((N, O), jnp.float32),
        ],
    )(parts, r, wlT, wrT, b.reshape(1, O))


def _final_body(parts_ref, cnts_ref, r_ref, out_ref):
    agg = parts_ref[0] + parts_ref[1]
    cnt = cnts_ref[0, :, H:H + 1] + cnts_ref[1, :, H:H + 1]
    out_ref[...] = agg / jnp.maximum(cnt, 1.0) + r_ref[...]


def _final_combine(parts, cnts, r):
    return pl.pallas_call(
        _final_body,
        grid=(N // RB,),
        in_specs=[
            pl.BlockSpec((NC, RB, O), lambda i: (0, i, 0)),
            pl.BlockSpec((NC, RB, W1), lambda i: (0, i, 0)),
            pl.BlockSpec((RB, O), lambda i: (i, 0)),
        ],
        out_specs=pl.BlockSpec((RB, O), lambda i: (i, 0)),
        out_shape=jax.ShapeDtypeStruct((N, O), jnp.float32),
    )(parts, cnts, r)


def _make_edge_scatter(d):
    mesh = plsc.VectorSubcoreMesh(core_axis_name="c", subcore_axis_name="s",
                                  num_cores=NC, num_subcores=NS)

    def body(table, e3d, zeros_d, out, idx_v, rows0, rows1, acc,
             sem0, sem1):
        cid = lax.axis_index("c")
        sid = lax.axis_index("s")
        tile = cid * NS + sid
        extra = tile < NX
        base = tile * RT + jnp.minimum(tile, NX)

        pltpu.sync_copy(e3d.at[:, pl.ds(base, RT)], idx_v.at[:, pl.ds(0, RT)])

        @pl.when(extra)
        def _():
            pltpu.sync_copy(e3d.at[:, pl.ds(base + RT, 1)],
                            idx_v.at[:, pl.ds(RT, 1)])

        pltpu.async_copy(table.at[idx_v.at[0, 0]], rows0, sem0)

        zlo = sid * ZR
        pltpu.sync_copy(zeros_d.at[pl.ds(zlo, ZR)], acc.at[pl.ds(zlo, ZR)])
        plsc.subcore_barrier()

        def pair(k, carry):
            j0 = 2 * k
            c1 = pltpu.async_copy(table.at[idx_v.at[0, j0 + 1]], rows1, sem1)
            pltpu.make_async_copy(table.at[idx_v.at[0, j0]], rows0,
                                  sem0).wait()
            pltpu.sync_copy(rows0, acc.at[idx_v.at[1, j0]], add=True)

            @pl.when((k < RT // 2 - 1) | extra)
            def _():
                pltpu.async_copy(table.at[idx_v.at[0, j0 + 2]], rows0, sem0)

            c1.wait()
            pltpu.sync_copy(rows1, acc.at[idx_v.at[1, j0 + 1]], add=True)
            return carry

        lax.fori_loop(0, RT // 2, pair, None)

        @pl.when(extra)
        def _():
            pltpu.make_async_copy(table.at[idx_v.at[0, RT]], rows0,
                                  sem0).wait()
            pltpu.sync_copy(rows0, acc.at[idx_v.at[1, RT]], add=True)

        plsc.subcore_barrier()

        pltpu.sync_copy(acc.at[pl.ds(zlo, ZR)],
                        out.at[cid, pl.ds(zlo, ZR)])

    return pl.kernel(
        body,
        out_type=[jax.ShapeDtypeStruct((NC, N, d), jnp.float32)],
        mesh=mesh,
        scratch_types=[
            pltpu.VMEM((2, RT + 1, 128), jnp.int32),
            pltpu.VMEM((128, d), jnp.float32),
            pltpu.VMEM((128, d), jnp.float32),
            pltpu.VMEM_SHARED((N, d), jnp.float32),
            pltpu.SemaphoreType.DMA,
            pltpu.SemaphoreType.DMA,
        ],
        compiler_params=pltpu.CompilerParams(use_tc_tiling_on_sc=False))


_edge_scatter_l1 = _make_edge_scatter(W1)
_edge_scatter_l2 = _make_edge_scatter(O)


def kernel(x, edge_index, W1l, b1, W1r, W2l, b2, W2r):
    e3d = edge_index.astype(jnp.int32).reshape(2, NR, 128)
    zeros_1 = jnp.zeros((N, W1), jnp.float32)
    zeros_2 = jnp.zeros((N, O), jnp.float32)

    p1, r1 = _project2(x, W1l.T, W1r.T, b1)
    part1, = _edge_scatter_l1(p1, e3d, zeros_1)
    p2, r2 = _combine_project2(part1, r1, W2l.T, W2r.T, b2)
    part2, = _edge_scatter_l2(p2, e3d, zeros_2)
    return _final_combine(part2, part1, r2)


# speedup vs baseline: 18.4798x; 1.1152x over previous
import jax
import jax.numpy as jnp
from jax import lax
from jax.experimental import pallas as pl
from jax.experimental.pallas import tpu as pltpu
from jax.experimental.pallas import tpu_sc as plsc

N = 10000
E = 320000
F = 128
H = 64
O = 32

NC, NS = 2, 16
NW = NC * NS
NR = E // 128
RT = NR // NW
NX = NR - RT * NW
ZR = N // NS
RB = 1000
W1 = H + 8


def _p1_body(x_ref, wl_ref, p_ref):
    mm = jnp.dot(x_ref[...], wl_ref[...], preferred_element_type=jnp.float32)
    p_ref[...] = jnp.concatenate(
        [mm, jnp.ones((RB, 1), jnp.float32), jnp.zeros((RB, 7), jnp.float32)],
        axis=1)


def _project_p1(x, wlT):
    return pl.pallas_call(
        _p1_body,
        grid=(N // RB,),
        in_specs=[
            pl.BlockSpec((RB, F), lambda i: (i, 0)),
            pl.BlockSpec((F, H), lambda i: (0, 0)),
        ],
        out_specs=pl.BlockSpec((RB, W1), lambda i: (i, 0)),
        out_shape=jax.ShapeDtypeStruct((N, W1), jnp.float32),
    )(x, wlT)


def _root_body(x_ref, w_ref, b_ref, r_ref):
    r_ref[...] = (jnp.dot(x_ref[...], w_ref[...],
                          preferred_element_type=jnp.float32) + b_ref[...])


def _project_root(x, wT, b, d_in, d_out):
    return pl.pallas_call(
        _root_body,
        grid=(N // RB,),
        in_specs=[
            pl.BlockSpec((RB, d_in), lambda i: (i, 0)),
            pl.BlockSpec((d_in, d_out), lambda i: (0, 0)),
            pl.BlockSpec((1, d_out), lambda i: (0, 0)),
        ],
        out_specs=pl.BlockSpec((RB, d_out), lambda i: (i, 0)),
        out_shape=jax.ShapeDtypeStruct((N, d_out), jnp.float32),
    )(x, wT, b.reshape(1, d_out))


def _combine_body(parts_ref, r_ref, wl_ref, p_ref, h_ref, ic_ref):
    agg = parts_ref[0, :, :H] + parts_ref[1, :, :H]
    cnt = parts_ref[0, :, H:H + 1] + parts_ref[1, :, H:H + 1]
    invc = 1.0 / jnp.maximum(cnt, 1.0)
    h = jnp.maximum(agg * invc + r_ref[...], 0.0)
    h_ref[...] = h
    ic_ref[...] = jnp.broadcast_to(invc, (RB, O))
    p_ref[...] = jnp.dot(h, wl_ref[...], preferred_element_type=jnp.float32)


def _combine_project(parts, r, wlT):
    return pl.pallas_call(
        _combine_body,
        grid=(N // RB,),
        in_specs=[
            pl.BlockSpec((NC, RB, 128), lambda i: (0, i, 0)),
            pl.BlockSpec((RB, H), lambda i: (i, 0)),
            pl.BlockSpec((H, O), lambda i: (0, 0)),
        ],
        out_specs=[
            pl.BlockSpec((RB, O), lambda i: (i, 0)),
            pl.BlockSpec((RB, H), lambda i: (i, 0)),
            pl.BlockSpec((RB, O), lambda i: (i, 0)),
        ],
        out_shape=[
            jax.ShapeDtypeStruct((N, O), jnp.float32),
            jax.ShapeDtypeStruct((N, H), jnp.float32),
            jax.ShapeDtypeStruct((N, O), jnp.float32),
        ],
    )(parts, r, wlT)


def _final_body(parts_ref, ic_ref, r_ref, out_ref):
    agg = parts_ref[0, :, :O] + parts_ref[1, :, :O]
    out_ref[...] = agg * ic_ref[...] + r_ref[...]


def _final_combine(parts, invc, r):
    return pl.pallas_call(
        _final_body,
        grid=(N // RB,),
        in_specs=[
            pl.BlockSpec((NC, RB, 128), lambda i: (0, i, 0)),
            pl.BlockSpec((RB, O), lambda i: (i, 0)),
            pl.BlockSpec((RB, O), lambda i: (i, 0)),
        ],
        out_specs=pl.BlockSpec((RB, O), lambda i: (i, 0)),
        out_shape=jax.ShapeDtypeStruct((N, O), jnp.float32),
    )(parts, invc, r)


def _make_edge_scatter(d):
    mesh = plsc.VectorSubcoreMesh(core_axis_name="c", subcore_axis_name="s",
                                  num_cores=NC, num_subcores=NS)

    def body(table, e3d, zeros_d, out, idx_v, rows0, rows1, acc,
             sem0, sem1):
        cid = lax.axis_index("c")
        sid = lax.axis_index("s")
        tile = cid * NS + sid
        extra = tile < NX
        base = tile * RT + jnp.minimum(tile, NX)

        pltpu.sync_copy(e3d.at[:, pl.ds(base, RT)], idx_v.at[:, pl.ds(0, RT)])

        @pl.when(extra)
        def _():
            pltpu.sync_copy(e3d.at[:, pl.ds(base + RT, 1)],
                            idx_v.at[:, pl.ds(RT, 1)])

        pltpu.async_copy(table.at[idx_v.at[0, 0]], rows0, sem0)

        zlo = sid * ZR
        pltpu.sync_copy(zeros_d.at[pl.ds(zlo, ZR)], acc.at[pl.ds(zlo, ZR)])
        plsc.subcore_barrier()

        def pair(k, carry):
            j0 = 2 * k
            c1 = pltpu.async_copy(table.at[idx_v.at[0, j0 + 1]], rows1, sem1)
            pltpu.make_async_copy(table.at[idx_v.at[0, j0]], rows0,
                                  sem0).wait()
            pltpu.sync_copy(rows0, acc.at[idx_v.at[1, j0]], add=True)

            @pl.when((k < RT // 2 - 1) | extra)
            def _():
                pltpu.async_copy(table.at[idx_v.at[0, j0 + 2]], rows0, sem0)

            c1.wait()
            pltpu.sync_copy(rows1, acc.at[idx_v.at[1, j0 + 1]], add=True)
            return carry

        lax.fori_loop(0, RT // 2, pair, None)

        @pl.when(extra)
        def _():
            pltpu.make_async_copy(table.at[idx_v.at[0, RT]], rows0,
                                  sem0).wait()
            pltpu.sync_copy(rows0, acc.at[idx_v.at[1, RT]], add=True)

        plsc.subcore_barrier()

        pltpu.sync_copy(acc.at[pl.ds(zlo, ZR)],
                        out.at[cid, pl.ds(zlo, ZR), pl.ds(0, d)])

    return pl.kernel(
        body,
        out_type=[jax.ShapeDtypeStruct((NC, N, 128), jnp.float32)],
        mesh=mesh,
        scratch_types=[
            pltpu.VMEM((2, RT + 1, 128), jnp.int32),
            pltpu.VMEM((128, d), jnp.float32),
            pltpu.VMEM((128, d), jnp.float32),
            pltpu.VMEM_SHARED((N, d), jnp.float32),
            pltpu.SemaphoreType.DMA,
            pltpu.SemaphoreType.DMA,
        ],
        compiler_params=pltpu.CompilerParams(use_tc_tiling_on_sc=False))


_edge_scatter_l1 = _make_edge_scatter(W1)
_edge_scatter_l2 = _make_edge_scatter(O)


def kernel(x, edge_index, W1l, b1, W1r, W2l, b2, W2r):
    e3d = edge_index.astype(jnp.int32).reshape(2, NR, 128)
    zeros_1 = jnp.zeros((N, W1), jnp.float32)
    zeros_2 = jnp.zeros((N, O), jnp.float32)

    p1 = _project_p1(x, W1l.T)
    part1, = _edge_scatter_l1(p1, e3d, zeros_1)
    r1 = _project_root(x, W1r.T, b1, F, H)
    p2, h, invc = _combine_project(part1, r1, W2l.T)
    part2, = _edge_scatter_l2(p2, e3d, zeros_2)
    r2 = _project_root(h, W2r.T, b2, H, O)
    return _final_combine(part2, invc, r2)


# speedup vs baseline: 21.2625x; 1.1506x over previous
import jax
import jax.numpy as jnp
from jax import lax
from jax.experimental import pallas as pl
from jax.experimental.pallas import tpu as pltpu
from jax.experimental.pallas import tpu_sc as plsc

N = 10000
E = 320000
F = 128
H = 64
O = 32

NC, NS = 2, 16
NW = NC * NS
NR = E // 128
RT = NR // NW
NX = NR - RT * NW
ZR = N // NS
RB = 1000
W1 = H + 8


def _p1_body(x_ref, wl_ref, p_ref):
    mm = jnp.dot(x_ref[...], wl_ref[...], preferred_element_type=jnp.float32)
    p_ref[...] = jnp.concatenate(
        [mm, jnp.ones((RB, 1), jnp.float32), jnp.zeros((RB, 7), jnp.float32)],
        axis=1)


def _project_p1(x, wlT):
    return pl.pallas_call(
        _p1_body,
        grid=(N // RB,),
        in_specs=[
            pl.BlockSpec((RB, F), lambda i: (i, 0)),
            pl.BlockSpec((F, H), lambda i: (0, 0)),
        ],
        out_specs=pl.BlockSpec((RB, W1), lambda i: (i, 0)),
        out_shape=jax.ShapeDtypeStruct((N, W1), jnp.float32),
    )(x, wlT)


def _root_body(x_ref, w_ref, b_ref, r_ref):
    r_ref[...] = (jnp.dot(x_ref[...], w_ref[...],
                          preferred_element_type=jnp.float32) + b_ref[...])


def _project_root(x, wT, b, d_in, d_out):
    return pl.pallas_call(
        _root_body,
        grid=(N // RB,),
        in_specs=[
            pl.BlockSpec((RB, d_in), lambda i: (i, 0)),
            pl.BlockSpec((d_in, d_out), lambda i: (0, 0)),
            pl.BlockSpec((1, d_out), lambda i: (0, 0)),
        ],
        out_specs=pl.BlockSpec((RB, d_out), lambda i: (i, 0)),
        out_shape=jax.ShapeDtypeStruct((N, d_out), jnp.float32),
    )(x, wT, b.reshape(1, d_out))


def _combine_body(parts_ref, r_ref, wl_ref, p_ref, h_ref, ic_ref):
    agg = parts_ref[0, :, :H] + parts_ref[1, :, :H]
    cnt = parts_ref[0, :, H:H + 1] + parts_ref[1, :, H:H + 1]
    invc = 1.0 / jnp.maximum(cnt, 1.0)
    h = jnp.maximum(agg * invc + r_ref[...], 0.0)
    h_ref[...] = h
    ic_ref[...] = jnp.broadcast_to(invc, (RB, O))
    p_ref[...] = jnp.dot(h, wl_ref[...], preferred_element_type=jnp.float32)


def _combine_project(parts, r, wlT):
    return pl.pallas_call(
        _combine_body,
        grid=(N // RB,),
        in_specs=[
            pl.BlockSpec((NC, RB, 128), lambda i: (0, i, 0)),
            pl.BlockSpec((RB, H), lambda i: (i, 0)),
            pl.BlockSpec((H, O), lambda i: (0, 0)),
        ],
        out_specs=[
            pl.BlockSpec((RB, O), lambda i: (i, 0)),
            pl.BlockSpec((RB, H), lambda i: (i, 0)),
            pl.BlockSpec((RB, O), lambda i: (i, 0)),
        ],
        out_shape=[
            jax.ShapeDtypeStruct((N, O), jnp.float32),
            jax.ShapeDtypeStruct((N, H), jnp.float32),
            jax.ShapeDtypeStruct((N, O), jnp.float32),
        ],
    )(parts, r, wlT)


def _final_body(parts_ref, ic_ref, r_ref, out_ref):
    agg = parts_ref[0, :, :O] + parts_ref[1, :, :O]
    out_ref[...] = agg * ic_ref[...] + r_ref[...]


def _final_combine(parts, invc, r):
    return pl.pallas_call(
        _final_body,
        grid=(N // RB,),
        in_specs=[
            pl.BlockSpec((NC, RB, 128), lambda i: (0, i, 0)),
            pl.BlockSpec((RB, O), lambda i: (i, 0)),
            pl.BlockSpec((RB, O), lambda i: (i, 0)),
        ],
        out_specs=pl.BlockSpec((RB, O), lambda i: (i, 0)),
        out_shape=jax.ShapeDtypeStruct((N, O), jnp.float32),
    )(parts, invc, r)


def _make_edge_scatter(d):
    mesh = plsc.VectorSubcoreMesh(core_axis_name="c", subcore_axis_name="s",
                                  num_cores=NC, num_subcores=NS)

    def body(table, e3d, zeros_d, out, idx_v, rows0, rows1, rows2, acc,
             sem0, sem1, sem2):
        cid = lax.axis_index("c")
        sid = lax.axis_index("s")
        tile = cid * NS + sid
        extra = tile < NX
        base = tile * RT + jnp.minimum(tile, NX)

        pltpu.sync_copy(e3d.at[:, pl.ds(base, RT)], idx_v.at[:, pl.ds(0, RT)])

        @pl.when(extra)
        def _():
            pltpu.sync_copy(e3d.at[:, pl.ds(base + RT, 1)],
                            idx_v.at[:, pl.ds(RT, 1)])

        pltpu.async_copy(table.at[idx_v.at[0, 0]], rows0, sem0)
        pltpu.async_copy(table.at[idx_v.at[0, 1]], rows1, sem1)

        zlo = sid * ZR
        pltpu.sync_copy(zeros_d.at[pl.ds(zlo, ZR)], acc.at[pl.ds(zlo, ZR)])
        plsc.subcore_barrier()

        nt = RT // 3

        def triple(k, carry):
            j = 3 * k
            pltpu.async_copy(table.at[idx_v.at[0, j + 2]], rows2, sem2)
            pltpu.make_async_copy(table.at[idx_v.at[0, j]], rows0,
                                  sem0).wait()
            pltpu.sync_copy(rows0, acc.at[idx_v.at[1, j]], add=True)

            @pl.when((k < nt - 1) | extra)
            def _():
                pltpu.async_copy(table.at[idx_v.at[0, j + 3]], rows0, sem0)

            pltpu.make_async_copy(table.at[idx_v.at[0, j + 1]], rows1,
                                  sem1).wait()
            pltpu.sync_copy(rows1, acc.at[idx_v.at[1, j + 1]], add=True)

            @pl.when(k < nt - 1)
            def _():
                pltpu.async_copy(table.at[idx_v.at[0, j + 4]], rows1, sem1)

            pltpu.make_async_copy(table.at[idx_v.at[0, j + 2]], rows2,
                                  sem2).wait()
            pltpu.sync_copy(rows2, acc.at[idx_v.at[1, j + 2]], add=True)
            return carry

        lax.fori_loop(0, nt, triple, None)

        @pl.when(extra)
        def _():
            pltpu.make_async_copy(table.at[idx_v.at[0, RT]], rows0,
                                  sem0).wait()
            pltpu.sync_copy(rows0, acc.at[idx_v.at[1, RT]], add=True)

        plsc.subcore_barrier()

        pltpu.sync_copy(acc.at[pl.ds(zlo, ZR)],
                        out.at[cid, pl.ds(zlo, ZR), pl.ds(0, d)])

    return pl.kernel(
        body,
        out_type=[jax.ShapeDtypeStruct((NC, N, 128), jnp.float32)],
        mesh=mesh,
        scratch_types=[
            pltpu.VMEM((2, RT + 1, 128), jnp.int32),
            pltpu.VMEM((128, d), jnp.float32),
            pltpu.VMEM((128, d), jnp.float32),
            pltpu.VMEM((128, d), jnp.float32),
            pltpu.VMEM_SHARED((N, d), jnp.float32),
            pltpu.SemaphoreType.DMA,
            pltpu.SemaphoreType.DMA,
            pltpu.SemaphoreType.DMA,
        ],
        compiler_params=pltpu.CompilerParams(use_tc_tiling_on_sc=False))


_edge_scatter_l1 = _make_edge_scatter(W1)
_edge_scatter_l2 = _make_edge_scatter(O)


def kernel(x, edge_index, W1l, b1, W1r, W2l, b2, W2r):
    e3d = edge_index.astype(jnp.int32).reshape(2, NR, 128)
    zeros_1 = jnp.zeros((N, W1), jnp.float32)
    zeros_2 = jnp.zeros((N, O), jnp.float32)

    p1 = _project_p1(x, W1l.T)
    part1, = _edge_scatter_l1(p1, e3d, zeros_1)
    r1 = _project_root(x, W1r.T, b1, F, H)
    p2, h, invc = _combine_project(part1, r1, W2l.T)
    part2, = _edge_scatter_l2(p2, e3d, zeros_2)
    r2 = _project_root(h, W2r.T, b2, H, O)
    return _final_combine(part2, invc, r2)


# speedup vs baseline: 22.2119x; 1.0447x over previous
import jax
import jax.numpy as jnp
import numpy as np
from jax import lax
from jax.experimental import pallas as pl
from jax.experimental.pallas import tpu as pltpu
from jax.experimental.pallas import tpu_sc as plsc

N = 10000
E = 320000
F = 128
H = 64
O = 32

NC, NS = 2, 16
NW = NC * NS
NU = E // 256
UT = NU // NW
ZR = N // NS
RB = 1000
W1 = H + 8


def _p1_body(x_ref, wl_ref, p_ref):
    mm = jnp.dot(x_ref[...], wl_ref[...], preferred_element_type=jnp.float32)
    p_ref[...] = jnp.concatenate(
        [mm, jnp.ones((RB, 1), jnp.float32), jnp.zeros((RB, 7), jnp.float32)],
        axis=1)


def _project_p1(x, wlT):
    return pl.pallas_call(
        _p1_body,
        grid=(N // RB,),
        in_specs=[
            pl.BlockSpec((RB, F), lambda i: (i, 0)),
            pl.BlockSpec((F, H), lambda i: (0, 0)),
        ],
        out_specs=pl.BlockSpec((RB, W1), lambda i: (i, 0)),
        out_shape=jax.ShapeDtypeStruct((N, W1), jnp.float32),
    )(x, wlT)


def _root_body(x_ref, w_ref, b_ref, r_ref):
    r_ref[...] = (jnp.dot(x_ref[...], w_ref[...],
                          preferred_element_type=jnp.float32) + b_ref[...])


def _project_root(x, wT, b, d_in, d_out):
    return pl.pallas_call(
        _root_body,
        grid=(N // RB,),
        in_specs=[
            pl.BlockSpec((RB, d_in), lambda i: (i, 0)),
            pl.BlockSpec((d_in, d_out), lambda i: (0, 0)),
            pl.BlockSpec((1, d_out), lambda i: (0, 0)),
        ],
        out_specs=pl.BlockSpec((RB, d_out), lambda i: (i, 0)),
        out_shape=jax.ShapeDtypeStruct((N, d_out), jnp.float32),
    )(x, wT, b.reshape(1, d_out))


def _combine_body(parts_ref, r_ref, wl_ref, p_ref, h_ref, ic_ref):
    agg = parts_ref[0, :, :H] + parts_ref[1, :, :H]
    cnt = parts_ref[0, :, H:H + 1] + parts_ref[1, :, H:H + 1]
    invc = 1.0 / jnp.maximum(cnt, 1.0)
    h = jnp.maximum(agg * invc + r_ref[...], 0.0)
    h_ref[...] = h
    ic_ref[...] = jnp.broadcast_to(invc, (RB, O))
    p_ref[...] = jnp.dot(h, wl_ref[...], preferred_element_type=jnp.float32)


def _combine_project(parts, r, wlT):
    return pl.pallas_call(
        _combine_body,
        grid=(N // RB,),
        in_specs=[
            pl.BlockSpec((NC, RB, 128), lambda i: (0, i, 0)),
            pl.BlockSpec((RB, H), lambda i: (i, 0)),
            pl.BlockSpec((H, O), lambda i: (0, 0)),
        ],
        out_specs=[
            pl.BlockSpec((RB, O), lambda i: (i, 0)),
            pl.BlockSpec((RB, H), lambda i: (i, 0)),
            pl.BlockSpec((RB, O), lambda i: (i, 0)),
        ],
        out_shape=[
            jax.ShapeDtypeStruct((N, O), jnp.float32),
            jax.ShapeDtypeStruct((N, H), jnp.float32),
            jax.ShapeDtypeStruct((N, O), jnp.float32),
        ],
    )(parts, r, wlT)


def _final_body(parts_ref, ic_ref, r_ref, out_ref):
    agg = parts_ref[0, :, :O] + parts_ref[1, :, :O]
    out_ref[...] = agg * ic_ref[...] + r_ref[...]


def _final_combine(parts, invc, r):
    return pl.pallas_call(
        _final_body,
        grid=(N // RB,),
        in_specs=[
            pl.BlockSpec((NC, RB, 128), lambda i: (0, i, 0)),
            pl.BlockSpec((RB, O), lambda i: (i, 0)),
            pl.BlockSpec((RB, O), lambda i: (i, 0)),
        ],
        out_specs=pl.BlockSpec((RB, O), lambda i: (i, 0)),
        out_shape=jax.ShapeDtypeStruct((N, O), jnp.float32),
    )(parts, invc, r)


def _make_edge_scatter(d):
    mesh = plsc.VectorSubcoreMesh(core_axis_name="c", subcore_axis_name="s",
                                  num_cores=NC, num_subcores=NS)

    def body(table, e3d, zeros_d, out, idx_v, rows0, rows1, rows2, acc,
             sem0, sem1, sem2):
        cid = lax.axis_index("c")
        sid = lax.axis_index("s")
        tile = cid * NS + sid
        extra = tile < NU - UT * NW
        base = tile * UT

        pltpu.sync_copy(e3d.at[:, pl.ds(base, UT)], idx_v.at[:, pl.ds(0, UT)])

        @pl.when(extra)
        def _():
            pltpu.sync_copy(e3d.at[:, pl.ds(UT * NW + tile, 1)],
                            idx_v.at[:, pl.ds(UT, 1)])

        pltpu.async_copy(table.at[idx_v.at[0, 0]], rows0, sem0)
        pltpu.async_copy(table.at[idx_v.at[0, 1]], rows1, sem1)

        zlo = sid * ZR
        pltpu.sync_copy(zeros_d.at[pl.ds(zlo, ZR)], acc.at[pl.ds(zlo, ZR)])
        plsc.subcore_barrier()

        nt = UT // 3

        def triple(k, carry):
            u = 3 * k
            pltpu.async_copy(table.at[idx_v.at[0, u + 2]], rows2, sem2)
            pltpu.make_async_copy(table.at[idx_v.at[0, u]], rows0,
                                  sem0).wait()
            pltpu.sync_copy(rows0, acc.at[idx_v.at[1, u]], add=True)

            @pl.when(k < nt - 1)
            def _():
                pltpu.async_copy(table.at[idx_v.at[0, u + 3]], rows0, sem0)

            pltpu.make_async_copy(table.at[idx_v.at[0, u + 1]], rows1,
                                  sem1).wait()
            pltpu.sync_copy(rows1, acc.at[idx_v.at[1, u + 1]], add=True)

            @pl.when(k < nt - 1)
            def _():
                pltpu.async_copy(table.at[idx_v.at[0, u + 4]], rows1, sem1)

            pltpu.make_async_copy(table.at[idx_v.at[0, u + 2]], rows2,
                                  sem2).wait()
            pltpu.sync_copy(rows2, acc.at[idx_v.at[1, u + 2]], add=True)
            return carry

        lax.fori_loop(0, nt, triple, None)

        @pl.when(extra)
        def _():
            pltpu.async_copy(table.at[idx_v.at[0, UT]], rows0, sem0)
            pltpu.make_async_copy(table.at[idx_v.at[0, UT]], rows0,
                                  sem0).wait()
            pltpu.sync_copy(rows0, acc.at[idx_v.at[1, UT]], add=True)

        plsc.subcore_barrier()

        pltpu.sync_copy(acc.at[pl.ds(zlo, ZR)],
                        out.at[cid, pl.ds(zlo, ZR), pl.ds(0, d)])

    return pl.kernel(
        body,
        out_type=[jax.ShapeDtypeStruct((NC, N, 128), jnp.float32)],
        mesh=mesh,
        scratch_types=[
            pltpu.VMEM((2, UT + 1, 256), jnp.int32),
            pltpu.VMEM((256, d), jnp.float32),
            pltpu.VMEM((256, d), jnp.float32),
            pltpu.VMEM((256, d), jnp.float32),
            pltpu.VMEM_SHARED((N, d), jnp.float32),
            pltpu.SemaphoreType.DMA,
            pltpu.SemaphoreType.DMA,
            pltpu.SemaphoreType.DMA,
        ],
        compiler_params=pltpu.CompilerParams(use_tc_tiling_on_sc=False))


_edge_scatter_l1 = _make_edge_scatter(W1)
_edge_scatter_l2 = _make_edge_scatter(O)


_ZEROS_1 = np.zeros((N, W1), np.float32)
_ZEROS_2 = np.zeros((N, O), np.float32)


def kernel(x, edge_index, W1l, b1, W1r, W2l, b2, W2r):
    e3d = edge_index.astype(jnp.int32).reshape(2, NU, 256)
    zeros_1 = _ZEROS_1
    zeros_2 = _ZEROS_2

    p1 = _project_p1(x, W1l.T)
    part1, = _edge_scatter_l1(p1, e3d, zeros_1)
    r1 = _project_root(x, W1r.T, b1, F, H)
    p2, h, invc = _combine_project(part1, r1, W2l.T)
    part2, = _edge_scatter_l2(p2, e3d, zeros_2)
    r2 = _project_root(h, W2r.T, b2, H, O)
    return _final_combine(part2, invc, r2)
